# baseline (speedup 1.0000x reference)
import functools

import jax
import jax.numpy as jnp
from jax import lax
from jax.experimental import pallas as pl
from jax.experimental.pallas import tpu as pltpu
from jax.experimental.pallas import tpu_sc as plsc

NC = 2
NS = 16
LANES = 16
NW = NC * NS

VOCAB = 1000000
BATCH = 4096
SEQ = 200
DIM = 64
CHUNK = 128
BCHUNKS = BATCH // CHUNK
NBLOCKS = SEQ * BCHUNKS
BL_PER_W = NBLOCKS // NW
KB = DIM // 8


def _body(idx_hbm, tab_hbm, pos_hbm, out_hbm, idx_v, h_v, g_v, w_v, pos_v,
          sem_g, sem_o):
    wid = lax.axis_index("s") * NC + lax.axis_index("c")
    blk0 = wid * BL_PER_W

    pltpu.sync_copy(idx_hbm.at[pl.ds(blk0 * CHUNK, BL_PER_W * CHUNK)], idx_v)
    pltpu.sync_copy(pos_hbm, pos_v)

    lanes = lax.iota(jnp.int32, LANES)
    rows16 = [lanes + g * LANES for g in range(CHUNK // LANES)]

    def build_and_fire(m, hb, gb):
        for g in range(CHUNK // LANES):
            iv = idx_v[pl.ds(m * CHUNK + g * LANES, LANES)]
            h_v.at[hb, pl.ds(g * LANES, LANES)][...] = lax.shift_right_logical(iv, 1)
        pltpu.async_copy(tab_hbm.at[h_v.at[hb]], g_v.at[gb], sem_g)

    def compute(m, b):
        q = blk0 + m
        s = q // BCHUNKS
        c = q % BCHUNKS
        p64 = [(idx_v[pl.ds(m * CHUNK + g * LANES, LANES)] &
                jnp.full((LANES,), 1, jnp.int32)) * jnp.full((LANES,), DIM, jnp.int32)
               for g in range(CHUNK // LANES)]
        s16 = jnp.full((LANES,), s, jnp.int32)

        def drow(d, carry):
            d16 = jnp.full((LANES,), d, jnp.int32)
            psp = plsc.load_gather(pos_v, [s16, d16])
            for g in range(CHUNK // LANES):
                col = p64[g] + d16
                v = plsc.load_gather(g_v.at[b], [rows16[g], col])
                w_v.at[b, d, pl.ds(g * LANES, LANES)][...] = v + psp
            return carry

        lax.fori_loop(0, DIM, drow, 0)

        for k in range(KB):
            pltpu.async_copy(w_v.at[b, pl.ds(k * 8, 8)], out_hbm.at[s, k, c], sem_o)

    def drain_w(b):
        for k in range(KB):
            pltpu.make_async_copy(w_v.at[b, pl.ds(k * 8, 8)],
                                  out_hbm.at[0, k, 0], sem_o).wait()

    def wait_g(b):
        pltpu.make_async_copy(tab_hbm.at[h_v.at[b]], g_v.at[b], sem_g).wait()

    build_and_fire(0, 0, 0)

    def step(it, carry):
        for sub in range(2):
            m = it * 2 + sub
            b = sub

            @pl.when(m + 1 < BL_PER_W)
            def _():
                build_and_fire(m + 1, 1 - b, 1 - b)

            wait_g(b)

            @pl.when(m >= 2)
            def _():
                drain_w(b)

            compute(m, b)
        return carry

    lax.fori_loop(0, BL_PER_W // 2, step, 0)
    drain_w(0)
    drain_w(1)


@jax.jit
def kernel(inputs, token_table, position_table):
    tab2 = token_table.reshape(VOCAB // 2, 2 * DIM)
    idx_flat = inputs.T.reshape(SEQ * BATCH)
    mesh = plsc.VectorSubcoreMesh(core_axis_name="c", subcore_axis_name="s")
    run = functools.partial(
        pl.kernel,
        out_type=jax.ShapeDtypeStruct((SEQ, KB, BCHUNKS, 8, CHUNK), jnp.float32),
        mesh=mesh,
        scratch_types=[
            pltpu.VMEM((BL_PER_W * CHUNK,), jnp.int32),
            pltpu.VMEM((2, CHUNK), jnp.int32),
            pltpu.VMEM((2, CHUNK, 2 * DIM), jnp.float32),
            pltpu.VMEM((2, DIM, CHUNK), jnp.float32),
            pltpu.VMEM((SEQ, DIM), jnp.float32),
            pltpu.SemaphoreType.DMA,
            pltpu.SemaphoreType.DMA,
        ],
        compiler_params=pltpu.CompilerParams(use_tc_tiling_on_sc=False,
                                             needs_layout_passes=False),
    )(_body)
    out5 = run(idx_flat, tab2, position_table)
    return out5.transpose(2, 4, 0, 1, 3).reshape(BATCH, SEQ, DIM)


# speedup vs baseline: 1.0894x; 1.0894x over previous
import functools

import jax
import jax.numpy as jnp
from jax import lax
from jax.experimental import pallas as pl
from jax.experimental.pallas import tpu as pltpu
from jax.experimental.pallas import tpu_sc as plsc

NC = 2
NS = 16
LANES = 16
NW = NC * NS

VOCAB = 1000000
BATCH = 4096
SEQ = 200
DIM = 64
CHUNK = 128
BCHUNKS = BATCH // CHUNK
NBLOCKS = SEQ * BCHUNKS
BL_PER_W = NBLOCKS // NW
KB = DIM // 8
WSTRIDE = CHUNK + 1


def _body(idx_hbm, tab_hbm, pos_hbm, out_hbm, idx_v, h_v, g_v, w1_v, wc_v,
          pos_v, sem_g, sem_o):
    wid = lax.axis_index("s") * NC + lax.axis_index("c")
    blk0 = wid * BL_PER_W

    pltpu.sync_copy(idx_hbm.at[pl.ds(blk0 * CHUNK, BL_PER_W * CHUNK)],
                    idx_v.at[pl.ds(0, BL_PER_W * CHUNK)])
    pltpu.sync_copy(pos_hbm, pos_v)

    lanes = lax.iota(jnp.int32, LANES)
    wbase = [(lanes + w * LANES) * jnp.full((LANES,), WSTRIDE, jnp.int32)
             for w in range(DIM // LANES)]

    def build_and_fire(m, hb, gb):
        for g in range(CHUNK // LANES):
            iv = idx_v[pl.ds(m * CHUNK + g * LANES, LANES)]
            h_v.at[hb, pl.ds(g * LANES, LANES)][...] = lax.shift_right_logical(iv, 1)
        pltpu.async_copy(tab_hbm.at[h_v.at[hb]], g_v.at[gb], sem_g)

    def compute(m, b):
        q = blk0 + m
        s = q // BCHUNKS
        c = q % BCHUNKS
        pr = [pos_v[s, pl.ds(w * LANES, LANES)] for w in range(DIM // LANES)]

        def row_body(r, carry):
            par = idx_v[pl.ds(m * CHUNK + r, LANES)][0] & 1
            off = par * DIM
            for w in range(DIM // LANES):
                v = g_v[b, r, pl.ds(off + w * LANES, LANES)] + pr[w]
                plsc.store_scatter(w1_v, [wbase[w] + r], v)
            return carry

        lax.fori_loop(0, CHUNK, row_body, 0)

        def pack_body(f, carry):
            for w in range(CHUNK // LANES):
                wc_v.at[b, f, pl.ds(w * LANES, LANES)][...] = \
                    w1_v[pl.ds(f * WSTRIDE + w * LANES, LANES)]
            return carry

        lax.fori_loop(0, DIM, pack_body, 0)

        for k in range(KB):
            pltpu.async_copy(wc_v.at[b, pl.ds(k * 8, 8)], out_hbm.at[s, k, c], sem_o)

    def drain_w(b):
        for k in range(KB):
            pltpu.make_async_copy(wc_v.at[b, pl.ds(k * 8, 8)],
                                  out_hbm.at[0, k, 0], sem_o).wait()

    def wait_g(b):
        pltpu.make_async_copy(tab_hbm.at[h_v.at[b]], g_v.at[b], sem_g).wait()

    build_and_fire(0, 0, 0)

    def step(it, carry):
        for sub in range(2):
            m = it * 2 + sub
            b = sub

            @pl.when(m + 1 < BL_PER_W)
            def _():
                build_and_fire(m + 1, 1 - b, 1 - b)

            wait_g(b)

            @pl.when(m >= 2)
            def _():
                drain_w(b)

            compute(m, b)
        return carry

    lax.fori_loop(0, BL_PER_W // 2, step, 0)
    drain_w(0)
    drain_w(1)


@jax.jit
def kernel(inputs, token_table, position_table):
    tab2 = token_table.reshape(VOCAB // 2, 2 * DIM)
    idx_flat = inputs.T.reshape(SEQ * BATCH)
    mesh = plsc.VectorSubcoreMesh(core_axis_name="c", subcore_axis_name="s")
    run = functools.partial(
        pl.kernel,
        out_type=jax.ShapeDtypeStruct((SEQ, KB, BCHUNKS, 8, CHUNK), jnp.float32),
        mesh=mesh,
        scratch_types=[
            pltpu.VMEM((BL_PER_W * CHUNK + LANES,), jnp.int32),
            pltpu.VMEM((2, CHUNK), jnp.int32),
            pltpu.VMEM((2, CHUNK, 2 * DIM), jnp.float32),
            pltpu.VMEM((DIM * WSTRIDE,), jnp.float32),
            pltpu.VMEM((2, DIM, CHUNK), jnp.float32),
            pltpu.VMEM((SEQ, DIM), jnp.float32),
            pltpu.SemaphoreType.DMA,
            pltpu.SemaphoreType.DMA,
        ],
        compiler_params=pltpu.CompilerParams(use_tc_tiling_on_sc=False,
                                             needs_layout_passes=False),
    )(_body)
    out5 = run(idx_flat, tab2, position_table)
    return out5.transpose(2, 4, 0, 1, 3).reshape(BATCH, SEQ, DIM)
